# baseline (speedup 1.0000x reference)
import functools

import jax
import jax.numpy as jnp
from jax import lax
from jax.experimental import pallas as pl
from jax.experimental.pallas import tpu as pltpu
from jax.experimental.pallas import tpu_sc as plsc

B = 128
V = 100000
E = 128
C = 64
H = 256
K = 64

VP = 100352
PAD = VP - V

_GUMBEL_PAD = jnp.pad(
    jax.random.gumbel(jax.random.key(1234), (B, V), dtype=jnp.float32),
    ((0, 0), (0, PAD)), constant_values=-jnp.inf)

ROWS = 8
NEG_INF = float("-inf")


def _topk_body(dist_ref, gum_ref, idx_ref):
    x = dist_ref[...] + gum_ref[...]
    iota = lax.broadcasted_iota(jnp.int32, (ROWS, VP), 1)

    def step(j, carry):
        x, acc = carry
        m = jnp.max(x, axis=1, keepdims=True)
        cand = jnp.where(x >= m, iota, VP)
        idx = jnp.min(cand, axis=1, keepdims=True)
        x = jnp.where(iota == idx, NEG_INF, x)
        lane = lax.broadcasted_iota(jnp.int32, (ROWS, K), 1)
        acc = jnp.where(lane == j, idx, acc)
        return x, acc

    _, acc = lax.fori_loop(0, K, step, (x, jnp.zeros((ROWS, K), jnp.int32)))
    idx_ref[...] = acc


def _topk(dist):
    dist_pad = jnp.pad(dist, ((0, 0), (0, PAD)), constant_values=-jnp.inf)
    return pl.pallas_call(
        _topk_body,
        grid=(B // ROWS,),
        in_specs=[
            pl.BlockSpec((ROWS, VP), lambda i: (i, 0)),
            pl.BlockSpec((ROWS, VP), lambda i: (i, 0)),
        ],
        out_specs=pl.BlockSpec((ROWS, K), lambda i: (i, 0)),
        out_shape=jax.ShapeDtypeStruct((B, K), jnp.int32),
    )(dist_pad, _GUMBEL_PAD)


_NC, _NS = 2, 16
_NW = _NC * _NS
N_IDS = B + B * K
N_IDS_PAD = 8448
B_PER_W = N_IDS_PAD // _NW


def _gather(table, idx):
    mesh = plsc.VectorSubcoreMesh(core_axis_name="c", subcore_axis_name="s")

    @functools.partial(
        pl.kernel, mesh=mesh,
        out_type=jax.ShapeDtypeStruct((N_IDS_PAD, C), jnp.float32),
        compiler_params=pltpu.CompilerParams(use_tc_tiling_on_sc=False),
        scratch_types=[
            pltpu.VMEM((B_PER_W,), jnp.int32),
            pltpu.VMEM((B_PER_W, C), jnp.float32),
            pltpu.SemaphoreType.DMA,
        ],
    )
    def gather_k(table_hbm, idx_hbm, out_hbm, idx_v, rows_v, sem):
        wid = lax.axis_index("s") * _NC + lax.axis_index("c")
        base = wid * B_PER_W
        pltpu.sync_copy(idx_hbm.at[pl.ds(base, B_PER_W)], idx_v)
        pltpu.async_copy(table_hbm.at[idx_v], rows_v, sem).wait()
        pltpu.sync_copy(rows_v, out_hbm.at[pl.ds(base, B_PER_W)])

    return gather_k(table, idx)


def _mlp_body(emb_ref, rows_ref, w1e_ref, w1c_ref, b1_ref, w2_ref, b2_ref,
              out_ref):
    emb = emb_ref[...]
    w1e = w1e_ref[...]
    w1c = w1c_ref[...]
    b1 = b1_ref[...]
    w2 = w2_ref[...]
    b2 = b2_ref[0, 0]

    a = jnp.dot(emb, w1e, preferred_element_type=jnp.float32)

    pos = rows_ref[0:B, :]
    cpos = jnp.dot(pos, w1c, preferred_element_type=jnp.float32)
    hpos = jnp.maximum(a + cpos + b1, 0.0)
    sp = jnp.dot(hpos, w2, preferred_element_type=jnp.float32) + b2

    neg = rows_ref[B:B + B * K, :]
    cneg = jnp.dot(neg, w1c, preferred_element_type=jnp.float32)
    a_rep = jnp.broadcast_to(a[:, None, :], (B, K, H)).reshape(B * K, H)
    hneg = jnp.maximum(a_rep + cneg + b1, 0.0)
    sn = jnp.dot(hneg, w2, preferred_element_type=jnp.float32) + b2

    tp = jnp.maximum(sp, 0.0) - sp + jnp.log(1.0 + jnp.exp(-jnp.abs(sp)))
    tn = jnp.maximum(sn, 0.0) + jnp.log(1.0 + jnp.exp(-jnp.abs(sn)))
    total = (jnp.sum(tp) + jnp.sum(tn)) / jnp.float32(B + B * K)
    out_ref[...] = jnp.reshape(total, (1, 1))


def _mlp_loss(embedding, rows, W1, b1, W2, b2):
    out = pl.pallas_call(
        _mlp_body,
        out_shape=jax.ShapeDtypeStruct((1, 1), jnp.float32),
    )(embedding, rows, W1[:E], W1[E:], b1.reshape(1, H), W2, b2.reshape(1, 1))
    return out.reshape(())


def kernel(embedding, gt_token, next_token_dist, emb_table, W1, b1, W2, b2):
    neg_idx = _topk(next_token_dist)
    ids = jnp.concatenate([gt_token.astype(jnp.int32), neg_idx.reshape(-1)])
    ids = jnp.pad(ids, (0, N_IDS_PAD - N_IDS))
    rows = _gather(emb_table, ids)
    return _mlp_loss(embedding, rows[:N_IDS], W1, b1, W2, b2)


# speedup vs baseline: 2.6789x; 2.6789x over previous
import functools

import jax
import jax.numpy as jnp
from jax import lax
from jax.experimental import pallas as pl
from jax.experimental.pallas import tpu as pltpu
from jax.experimental.pallas import tpu_sc as plsc

B = 128
V = 100000
E = 128
C = 64
H = 256
K = 64

_GUMBEL = jax.random.gumbel(jax.random.key(1234), (B, V), dtype=jnp.float32)

NC, NS = 2, 16
NW = NC * NS
RPW = B // NW
CH = 10000
NCH = V // CH
NVR = CH // 16
BATCH = 8
NBATCH = (NVR - 1) // BATCH
CAP = 1024
COMPACT_AT = 640
GROUP = 1 + K
N_IDS = B * GROUP
IPW = RPW * GROUP


def _key16(v):
    u = lax.bitcast_convert_type(v, jnp.uint32)
    return jnp.where(u >= jnp.uint32(0x80000000), ~u,
                     u | jnp.uint32(0x80000000))


def _unkey(k):
    u = jnp.where(k >= jnp.uint32(0x80000000),
                  k ^ jnp.uint32(0x80000000), ~k)
    return lax.bitcast_convert_type(u, jnp.float32)


def _sc_topk_gather(dist_flat, gumbel_flat, gt, table):
    mesh = plsc.VectorSubcoreMesh(core_axis_name="c", subcore_axis_name="s")

    @functools.partial(
        pl.kernel, mesh=mesh,
        out_type=jax.ShapeDtypeStruct((N_IDS, C), jnp.float32),
        compiler_params=pltpu.CompilerParams(use_tc_tiling_on_sc=False,
                                             needs_layout_passes=False),
        scratch_types=[
            pltpu.VMEM((CH,), jnp.float32),
            pltpu.VMEM((CH,), jnp.float32),
            pltpu.VMEM((CAP,), jnp.float32),
            pltpu.VMEM((CAP,), jnp.int32),
            pltpu.VMEM((IPW,), jnp.int32),
            pltpu.VMEM((IPW, C), jnp.float32),
            pltpu.VMEM((16,), jnp.int32),
            pltpu.SemaphoreType.DMA,
        ],
    )
    def body(dist_hbm, gum_hbm, gt_hbm, table_hbm, out_hbm,
             dist_v, gum_v, candv, candi, idxg, rows_v, gt_v, sem):
        wid = lax.axis_index("s") * NC + lax.axis_index("c")
        lane = lax.broadcasted_iota(jnp.int32, (16,), 0)

        pltpu.sync_copy(gt_hbm.at[pl.ds(16 * (wid // 4), 16)], gt_v)
        g16 = gt_v[...]
        sub = 4 * (wid % 4)
        maskg = (lane >= sub) & (lane < sub + 4)
        posg = jnp.where(maskg, (lane - sub) * GROUP, 0)
        plsc.store_scatter(idxg, [posg], g16, mask=maskg)

        def cnt_gt(ms, count):
            nv = (count + 15) // 16

            def cbody(v, ctr):
                key = _key16(candv[pl.ds(16 * v, 16)])
                valid = (lane + 16 * v) < count
                return ctr + jnp.sum(jnp.where((key > ms) & valid, 1, 0))

            return lax.fori_loop(0, nv, cbody, jnp.int32(0))

        def compact(ct):
            count, tau = ct
            nv = (count + 15) // 16

            def bs_body(_, lohi):
                lo, hi = lohi
                mid = lo + (hi - lo) // jnp.uint32(2)
                big = cnt_gt(mid, count) >= 64
                return (jnp.where(big, mid, lo), jnp.where(big, hi, mid))

            lo, _ = lax.fori_loop(
                0, 12, bs_body,
                (jnp.uint32(0), jnp.uint32(0xFFFFFFFF)))

            def rbody(v, nc):
                val = candv[pl.ds(16 * v, 16)]
                ivv = candi[pl.ds(16 * v, 16)]
                keep = (_key16(val) > lo) & ((lane + 16 * v) < count)
                ones = jnp.where(keep, 1, 0)
                pref = plsc.cumsum(ones)
                pos = jnp.where(keep, nc + pref - 1, 0)
                plsc.store_scatter(candv, [pos], val, mask=keep)
                plsc.store_scatter(candi, [pos], ivv, mask=keep)
                return nc + jnp.sum(ones)

            newcount = lax.fori_loop(0, nv, rbody, jnp.int32(0))
            return newcount, jnp.maximum(tau, _unkey(lo))

        def maybe_compact(count, tau):
            return lax.cond(count >= COMPACT_AT, compact,
                            lambda ct: ct, (count, tau))

        def append_vregs(xs, ibase, count, tau):
            for i, x in enumerate(xs):
                m = x > tau
                ones = jnp.where(m, 1, 0)
                pref = plsc.cumsum(ones)
                pos = count + pref - 1
                okm = m & (pos < CAP)
                pos = jnp.where(okm, pos, 0)
                plsc.store_scatter(candv, [pos], x, mask=okm)
                plsc.store_scatter(candi, [pos], ibase + 16 * i + lane, mask=okm)
                count = jnp.minimum(count + jnp.sum(ones), CAP)
            return maybe_compact(count, tau)

        def row_body(j, _):
            row = wid * RPW + j

            def chunk_body(c, ct):
                count, tau = ct
                off = row * V + c * CH
                pltpu.sync_copy(dist_hbm.at[pl.ds(off, CH)], dist_v)
                pltpu.sync_copy(gum_hbm.at[pl.ds(off, CH)], gum_v)
                gbase = c * CH

                def batch_body(bb, ct):
                    count, tau = ct
                    base = bb * (16 * BATCH)
                    xs = [dist_v[pl.ds(base + 16 * i, 16)]
                          + gum_v[pl.ds(base + 16 * i, 16)]
                          for i in range(BATCH)]
                    acc = xs[0]
                    for x in xs[1:]:
                        acc = jnp.maximum(acc, x)
                    hit = jnp.max(acc) > tau
                    return lax.cond(
                        hit,
                        lambda ct: append_vregs(xs, gbase + base, *ct),
                        lambda ct: ct, (count, tau))

                count, tau = lax.fori_loop(0, NBATCH, batch_body,
                                           (count, tau))
                tbase = NBATCH * 16 * BATCH
                xt = dist_v[pl.ds(tbase, 16)] + gum_v[pl.ds(tbase, 16)]
                count, tau = lax.cond(
                    jnp.max(xt) > tau,
                    lambda ct: append_vregs([xt], gbase + tbase, *ct),
                    lambda ct: ct, (count, tau))
                return count, tau

            count, tau = lax.fori_loop(
                0, NCH, chunk_body,
                (jnp.int32(0), jnp.float32(-jnp.inf)))

            def bs2(_, lohi):
                lo, hi = lohi
                mid = lo + (hi - lo) // jnp.uint32(2)
                le = cnt_gt(mid, count) <= 63
                return (jnp.where(le, lo, mid + jnp.uint32(1)),
                        jnp.where(le, mid, hi))

            _, tstar = lax.fori_loop(
                0, 32, bs2, (jnp.uint32(0), jnp.uint32(0xFFFFFFFF)))
            cg_total = cnt_gt(tstar, count)
            obase = j * GROUP + 1
            nv = (count + 15) // 16

            def ebody(v, cnts):
                cg, ce = cnts
                val = candv[pl.ds(16 * v, 16)]
                ivv = candi[pl.ds(16 * v, 16)]
                key = _key16(val)
                valid = (lane + 16 * v) < count
                gtm = (key > tstar) & valid
                eqm = (key == tstar) & valid
                og = jnp.where(gtm, 1, 0)
                oe = jnp.where(eqm, 1, 0)
                pg = obase + cg + plsc.cumsum(og) - 1
                pe = obase + cg_total + ce + plsc.cumsum(oe) - 1
                pe_ok = eqm & (pe < obase + K)
                plsc.store_scatter(idxg, [jnp.where(gtm, pg, 0)], ivv, mask=gtm)
                plsc.store_scatter(idxg, [jnp.where(pe_ok, pe, 0)], ivv,
                                   mask=pe_ok)
                return cg + jnp.sum(og), ce + jnp.sum(oe)

            lax.fori_loop(0, nv, ebody, (jnp.int32(0), jnp.int32(0)))
            return 0

        lax.fori_loop(0, RPW, row_body, 0)

        pltpu.async_copy(table_hbm.at[idxg], rows_v, sem).wait()
        pltpu.sync_copy(rows_v, out_hbm.at[pl.ds(wid * IPW, IPW)])

    return body(dist_flat, gumbel_flat, gt, table)


def _mlp_body(emb_ref, rows_ref, w1e_ref, w1c_ref, b1_ref, w2_ref, b2_ref,
              out_ref):
    emb = emb_ref[...]
    w1e = w1e_ref[...]
    w1c = w1c_ref[...]
    b1 = b1_ref[...]
    w2 = w2_ref[...]
    b2 = b2_ref[0, 0]

    a = jnp.dot(emb, w1e, preferred_element_type=jnp.float32)
    ctx = rows_ref[...]
    cc = jnp.dot(ctx, w1c, preferred_element_type=jnp.float32)
    a_rep = jnp.broadcast_to(a[:, None, :], (B, GROUP, H)).reshape(N_IDS, H)
    h = jnp.maximum(a_rep + cc + b1, 0.0)
    s = jnp.dot(h, w2, preferred_element_type=jnp.float32) + b2

    ridx = lax.broadcasted_iota(jnp.int32, (N_IDS, 1), 0)
    y = jnp.where(ridx % GROUP == 0, 1.0, 0.0)
    t = jnp.maximum(s, 0.0) - s * y + jnp.log(1.0 + jnp.exp(-jnp.abs(s)))
    out_ref[...] = jnp.reshape(jnp.sum(t) / jnp.float32(N_IDS), (1, 1))


def _mlp_loss(embedding, rows, W1, b1, W2, b2):
    out = pl.pallas_call(
        _mlp_body,
        out_shape=jax.ShapeDtypeStruct((1, 1), jnp.float32),
    )(embedding, rows, W1[:E], W1[E:], b1.reshape(1, H), W2, b2.reshape(1, 1))
    return out.reshape(())


def kernel(embedding, gt_token, next_token_dist, emb_table, W1, b1, W2, b2):
    ctx = _sc_topk_gather(next_token_dist.reshape(-1), _GUMBEL.reshape(-1),
                          gt_token.astype(jnp.int32), emb_table)
    return _mlp_loss(embedding, ctx, W1, b1, W2, b2)


# speedup vs baseline: 7.3550x; 2.7456x over previous
import functools

import jax
import jax.numpy as jnp
from jax import lax
from jax.experimental import pallas as pl
from jax.experimental.pallas import tpu as pltpu
from jax.experimental.pallas import tpu_sc as plsc

B = 128
V = 100000
E = 128
C = 64
H = 256
K = 64

_GUMBEL = jax.random.gumbel(jax.random.key(1234), (B, V), dtype=jnp.float32)

NC, NS = 2, 16
NW = NC * NS
RPW = B // NW
CH = 5888
NCH = 17
BATCH = 16
NBATCH = CH // 16 // BATCH
CAP = 1024
COMPACT_AT = 640
GROUP = 1 + K
N_IDS = B * GROUP
IPW = RPW * GROUP
IPW_PAD = 264
N_IDS_PAD = NW * IPW_PAD


def _key16(v):
    u = lax.bitcast_convert_type(v, jnp.uint32)
    return jnp.where(u >= jnp.uint32(0x80000000), ~u,
                     u | jnp.uint32(0x80000000))


def _unkey(k):
    u = jnp.where(k >= jnp.uint32(0x80000000),
                  k ^ jnp.uint32(0x80000000), ~k)
    return lax.bitcast_convert_type(u, jnp.float32)


def _sc_topk(dist, gumbel, gt):
    mesh = plsc.VectorSubcoreMesh(core_axis_name="c", subcore_axis_name="s")

    @functools.partial(
        pl.kernel, mesh=mesh,
        out_type=jax.ShapeDtypeStruct((N_IDS_PAD,), jnp.int32),
        compiler_params=pltpu.CompilerParams(use_tc_tiling_on_sc=True,
                                             needs_layout_passes=False),
        scratch_types=[
            pltpu.VMEM((CH,), jnp.float32),
            pltpu.VMEM((CH,), jnp.float32),
            pltpu.VMEM((CH,), jnp.float32),
            pltpu.VMEM((CH,), jnp.float32),
            pltpu.VMEM((CAP,), jnp.float32),
            pltpu.VMEM((CAP,), jnp.int32),
            pltpu.VMEM((IPW_PAD,), jnp.int32),
            pltpu.VMEM((16,), jnp.int32),
            pltpu.SemaphoreType.DMA,
            pltpu.SemaphoreType.DMA,
            pltpu.SemaphoreType.DMA,
            pltpu.SemaphoreType.DMA,
        ],
    )
    def body(dist_hbm, gum_hbm, gt_hbm, ids_hbm,
             dist_v0, dist_v1, gum_v0, gum_v1, candv, candi, idxg,
             gt_v, semd0, semd1, semg0, semg1):
        wid = lax.axis_index("s") * NC + lax.axis_index("c")
        lane = lax.broadcasted_iota(jnp.int32, (16,), 0)

        pltpu.sync_copy(gt_hbm.at[pl.ds(16 * (wid // 4), 16)], gt_v)
        g16 = gt_v[...]
        sub = 4 * (wid % 4)
        maskg = (lane >= sub) & (lane < sub + 4)
        posg = jnp.where(maskg, (lane - sub) * GROUP, 0)
        plsc.store_scatter(idxg, [posg], g16, mask=maskg)
        plsc.store_scatter(idxg, [IPW + lane], jnp.zeros((16,), jnp.int32),
                           mask=lane < IPW_PAD - IPW)

        def cnt_gt(ms, count):
            nv = (count + 15) // 16

            def cbody(v, ctr):
                key = _key16(candv[pl.ds(16 * v, 16)])
                valid = (lane + 16 * v) < count
                return ctr + jnp.sum(jnp.where((key > ms) & valid, 1, 0))

            return lax.fori_loop(0, nv, cbody, jnp.int32(0))

        def compact(ct):
            count, tau = ct
            nv = (count + 15) // 16

            def bs_body(_, lohi):
                lo, hi = lohi
                mid = lo + (hi - lo) // jnp.uint32(2)
                big = cnt_gt(mid, count) >= 64
                return (jnp.where(big, mid, lo), jnp.where(big, hi, mid))

            lo, _ = lax.fori_loop(
                0, 12, bs_body,
                (jnp.uint32(0), jnp.uint32(0xFFFFFFFF)))

            def rbody(v, nc):
                val = candv[pl.ds(16 * v, 16)]
                ivv = candi[pl.ds(16 * v, 16)]
                keep = (_key16(val) > lo) & ((lane + 16 * v) < count)
                ones = jnp.where(keep, 1, 0)
                pref = plsc.cumsum(ones)
                pos = jnp.where(keep, nc + pref - 1, 0)
                plsc.store_scatter(candv, [pos], val, mask=keep)
                plsc.store_scatter(candi, [pos], ivv, mask=keep)
                return nc + jnp.sum(ones)

            newcount = lax.fori_loop(0, nv, rbody, jnp.int32(0))
            return newcount, jnp.maximum(tau, _unkey(lo))

        def maybe_compact(count, tau):
            return lax.cond(count >= COMPACT_AT, compact,
                            lambda ct: ct, (count, tau))

        def append_vregs(xs, ibase, count, tau):
            for i, x in enumerate(xs):
                m = x > tau
                ones = jnp.where(m, 1, 0)
                pref = plsc.cumsum(ones)
                pos = count + pref - 1
                okm = m & (pos < CAP)
                pos = jnp.where(okm, pos, 0)
                plsc.store_scatter(candv, [pos], x, mask=okm)
                plsc.store_scatter(candi, [pos], ibase + 16 * i + lane,
                                   mask=okm)
                count = jnp.minimum(count + jnp.sum(ones), CAP)
            return maybe_compact(count, tau)

        def row_body(j, _):
            row = wid * RPW + j

            def dma_pair(c, dv, gv, sd, sg):
                return (pltpu.make_async_copy(
                            dist_hbm.at[row, pl.ds(c * CH, CH)], dv, sd),
                        pltpu.make_async_copy(
                            gum_hbm.at[row, pl.ds(c * CH, CH)], gv, sg))

            def start_chunk(c, dv, gv, sd, sg):
                a, b2 = dma_pair(c, dv, gv, sd, sg)
                a.start()
                b2.start()

            def wait_chunk(c, dv, gv, sd, sg):
                a, b2 = dma_pair(c, dv, gv, sd, sg)
                a.wait()
                b2.wait()

            def one_batch(dv, gv, gbase, base, count, tau, masked):
                xs = [dv[pl.ds(base + 16 * i, 16)]
                      + gv[pl.ds(base + 16 * i, 16)]
                      for i in range(BATCH)]
                if masked:
                    xs = [jnp.where(gbase + base + 16 * i + lane < V, x,
                                    jnp.float32(-jnp.inf))
                          for i, x in enumerate(xs)]
                acc = xs[0]
                for x in xs[1:]:
                    acc = jnp.maximum(acc, x)
                hit = jnp.max(acc) > tau
                return lax.cond(
                    hit,
                    lambda ct: append_vregs(xs, gbase + base, *ct),
                    lambda ct: ct, (count, tau))

            def proc(dv, gv, c, count, tau, nb=NBATCH):
                gbase = c * CH

                def batch_body(bb, ct):
                    return one_batch(dv, gv, gbase, bb * (16 * BATCH),
                                     *ct, masked=False)

                return lax.fori_loop(0, nb, batch_body, (count, tau))

            start_chunk(0, dist_v0, gum_v0, semd0, semg0)

            def pair_body(cc, ct):
                count, tau = ct
                c0 = 2 * cc
                wait_chunk(c0, dist_v0, gum_v0, semd0, semg0)
                start_chunk(c0 + 1, dist_v1, gum_v1, semd1, semg1)
                count, tau = proc(dist_v0, gum_v0, c0, count, tau)
                wait_chunk(c0 + 1, dist_v1, gum_v1, semd1, semg1)
                start_chunk(c0 + 2, dist_v0, gum_v0, semd0, semg0)
                return proc(dist_v1, gum_v1, c0 + 1, count, tau)

            count, tau = lax.fori_loop(
                0, NCH // 2, pair_body,
                (jnp.int32(0), jnp.float32(-jnp.inf)))

            cl = jnp.int32(NCH - 1)
            wait_chunk(cl, dist_v0, gum_v0, semd0, semg0)
            count, tau = proc(dist_v0, gum_v0, cl, count, tau, nb=NBATCH - 1)
            count, tau = one_batch(dist_v0, gum_v0, cl * CH,
                                   (NBATCH - 1) * 16 * BATCH, count, tau,
                                   masked=True)

            def bs2(_, lohi):
                lo, hi = lohi
                mid = lo + (hi - lo) // jnp.uint32(2)
                le = cnt_gt(mid, count) <= 63
                return (jnp.where(le, lo, mid + jnp.uint32(1)),
                        jnp.where(le, mid, hi))

            _, tstar = lax.fori_loop(
                0, 32, bs2, (jnp.uint32(0), jnp.uint32(0xFFFFFFFF)))
            cg_total = cnt_gt(tstar, count)
            obase = j * GROUP + 1
            nv = (count + 15) // 16

            def ebody(v, cnts):
                cg, ce = cnts
                val = candv[pl.ds(16 * v, 16)]
                ivv = candi[pl.ds(16 * v, 16)]
                key = _key16(val)
                valid = (lane + 16 * v) < count
                gtm = (key > tstar) & valid
                eqm = (key == tstar) & valid
                og = jnp.where(gtm, 1, 0)
                oe = jnp.where(eqm, 1, 0)
                pg = obase + cg + plsc.cumsum(og) - 1
                pe = obase + cg_total + ce + plsc.cumsum(oe) - 1
                pe_ok = eqm & (pe < obase + K)
                plsc.store_scatter(idxg, [jnp.where(gtm, pg, 0)], ivv,
                                   mask=gtm)
                plsc.store_scatter(idxg, [jnp.where(pe_ok, pe, 0)], ivv,
                                   mask=pe_ok)
                return cg + jnp.sum(og), ce + jnp.sum(oe)

            lax.fori_loop(0, nv, ebody, (jnp.int32(0), jnp.int32(0)))
            return 0

        lax.fori_loop(0, RPW, row_body, 0)
        pltpu.sync_copy(idxg, ids_hbm.at[pl.ds(wid * IPW_PAD, IPW_PAD)])

    return body(dist, gumbel, gt)


def _sc_gather(table, ids):
    mesh = plsc.VectorSubcoreMesh(core_axis_name="c", subcore_axis_name="s")

    @functools.partial(
        pl.kernel, mesh=mesh,
        out_type=jax.ShapeDtypeStruct((N_IDS, C), jnp.float32),
        compiler_params=pltpu.CompilerParams(use_tc_tiling_on_sc=False,
                                             needs_layout_passes=False),
        scratch_types=[
            pltpu.VMEM((IPW_PAD,), jnp.int32),
            pltpu.VMEM((IPW, C), jnp.float32),
            pltpu.SemaphoreType.DMA,
        ],
    )
    def gather_k(table_hbm, ids_hbm, out_hbm, idx_v, rows_v, sem):
        wid = lax.axis_index("s") * NC + lax.axis_index("c")
        pltpu.sync_copy(ids_hbm.at[pl.ds(wid * IPW_PAD, IPW_PAD)], idx_v)
        pltpu.async_copy(table_hbm.at[idx_v.at[pl.ds(0, IPW)]], rows_v,
                         sem).wait()
        pltpu.sync_copy(rows_v, out_hbm.at[pl.ds(wid * IPW, IPW)])

    return gather_k(table, ids)


def _mlp_body(emb_ref, rows_ref, w1e_ref, w1c_ref, b1_ref, w2_ref, b2_ref,
              out_ref):
    emb = emb_ref[...]
    w1e = w1e_ref[...]
    w1c = w1c_ref[...]
    b1 = b1_ref[...]
    w2 = w2_ref[...]
    b2 = b2_ref[0, 0]

    a = jnp.dot(emb, w1e, preferred_element_type=jnp.float32)
    ctx = rows_ref[...]
    cc = jnp.dot(ctx, w1c, preferred_element_type=jnp.float32)
    a_rep = jnp.broadcast_to(a[:, None, :], (B, GROUP, H)).reshape(N_IDS, H)
    h = jnp.maximum(a_rep + cc + b1, 0.0)
    s = jnp.dot(h, w2, preferred_element_type=jnp.float32) + b2

    ridx = lax.broadcasted_iota(jnp.int32, (N_IDS, 1), 0)
    y = jnp.where(ridx % GROUP == 0, 1.0, 0.0)
    t = jnp.maximum(s, 0.0) - s * y + jnp.log(1.0 + jnp.exp(-jnp.abs(s)))
    out_ref[...] = jnp.reshape(jnp.sum(t) / jnp.float32(N_IDS), (1, 1))


def _mlp_loss(embedding, rows, W1, b1, W2, b2):
    out = pl.pallas_call(
        _mlp_body,
        out_shape=jax.ShapeDtypeStruct((1, 1), jnp.float32),
    )(embedding, rows, W1[:E], W1[E:], b1.reshape(1, H), W2, b2.reshape(1, 1))
    return out.reshape(())


def kernel(embedding, gt_token, next_token_dist, emb_table, W1, b1, W2, b2):
    ids = _sc_topk(next_token_dist, _GUMBEL, gt_token.astype(jnp.int32))
    ctx = _sc_gather(emb_table, ids)
    return _mlp_loss(embedding, ctx, W1, b1, W2, b2)


# speedup vs baseline: 7.8403x; 1.0660x over previous
import functools

import jax
import jax.numpy as jnp
from jax import lax
from jax.experimental import pallas as pl
from jax.experimental.pallas import tpu as pltpu
from jax.experimental.pallas import tpu_sc as plsc

B = 128
V = 100000
E = 128
C = 64
H = 256
K = 64

_GUMBEL = jax.random.gumbel(jax.random.key(1234), (B, V), dtype=jnp.float32)

NC, NS = 2, 16
NW = NC * NS
RPW = B // NW
CH = 5888
NCH = 17
BATCH = 23
NBATCH = CH // 16 // BATCH
CAP = 1024
COMPACT_AT = 640
GROUP = 1 + K
N_IDS = B * GROUP
IPW = RPW * GROUP
IPW_PAD = 264
N_IDS_PAD = NW * IPW_PAD


def _key16(v):
    u = lax.bitcast_convert_type(v, jnp.uint32)
    return jnp.where(u >= jnp.uint32(0x80000000), ~u,
                     u | jnp.uint32(0x80000000))


def _unkey(k):
    u = jnp.where(k >= jnp.uint32(0x80000000),
                  k ^ jnp.uint32(0x80000000), ~k)
    return lax.bitcast_convert_type(u, jnp.float32)


def _sc_topk(dist, gumbel, gt):
    mesh = plsc.VectorSubcoreMesh(core_axis_name="c", subcore_axis_name="s")

    @functools.partial(
        pl.kernel, mesh=mesh,
        out_type=jax.ShapeDtypeStruct((N_IDS_PAD,), jnp.int32),
        compiler_params=pltpu.CompilerParams(use_tc_tiling_on_sc=True,
                                             needs_layout_passes=False),
        scratch_types=[
            pltpu.VMEM((CH,), jnp.float32),
            pltpu.VMEM((CH,), jnp.float32),
            pltpu.VMEM((CH,), jnp.float32),
            pltpu.VMEM((CH,), jnp.float32),
            pltpu.VMEM((CAP,), jnp.float32),
            pltpu.VMEM((CAP,), jnp.int32),
            pltpu.VMEM((IPW_PAD,), jnp.int32),
            pltpu.VMEM((16,), jnp.int32),
            pltpu.SemaphoreType.DMA,
            pltpu.SemaphoreType.DMA,
            pltpu.SemaphoreType.DMA,
            pltpu.SemaphoreType.DMA,
        ],
    )
    def body(dist_hbm, gum_hbm, gt_hbm, ids_hbm,
             dist_v0, dist_v1, gum_v0, gum_v1, candv, candi, idxg,
             gt_v, semd0, semd1, semg0, semg1):
        wid = lax.axis_index("s") * NC + lax.axis_index("c")
        lane = lax.broadcasted_iota(jnp.int32, (16,), 0)

        pltpu.sync_copy(gt_hbm.at[pl.ds(16 * (wid // 4), 16)], gt_v)
        g16 = gt_v[...]
        sub = 4 * (wid % 4)
        maskg = (lane >= sub) & (lane < sub + 4)
        posg = jnp.where(maskg, (lane - sub) * GROUP, 0)
        plsc.store_scatter(idxg, [posg], g16, mask=maskg)
        plsc.store_scatter(idxg, [IPW + lane], jnp.zeros((16,), jnp.int32),
                           mask=lane < IPW_PAD - IPW)

        def cnt_gt(ms, count):
            nv = (count + 15) // 16

            def cbody(v, ctr):
                key = _key16(candv[pl.ds(16 * v, 16)])
                valid = (lane + 16 * v) < count
                return ctr + jnp.sum(jnp.where((key > ms) & valid, 1, 0))

            return lax.fori_loop(0, nv, cbody, jnp.int32(0))

        def compact(ct):
            count, tau = ct
            nv = (count + 15) // 16

            def bs_body(_, lohi):
                lo, hi = lohi
                mid = lo + (hi - lo) // jnp.uint32(2)
                big = cnt_gt(mid, count) >= 64
                return (jnp.where(big, mid, lo), jnp.where(big, hi, mid))

            lo, _ = lax.fori_loop(
                0, 12, bs_body,
                (jnp.uint32(0), jnp.uint32(0xFFFFFFFF)))

            def rbody(v, nc):
                val = candv[pl.ds(16 * v, 16)]
                ivv = candi[pl.ds(16 * v, 16)]
                keep = (_key16(val) > lo) & ((lane + 16 * v) < count)
                ones = jnp.where(keep, 1, 0)
                pref = plsc.cumsum(ones)
                pos = jnp.where(keep, nc + pref - 1, 0)
                plsc.store_scatter(candv, [pos], val, mask=keep)
                plsc.store_scatter(candi, [pos], ivv, mask=keep)
                return nc + jnp.sum(ones)

            newcount = lax.fori_loop(0, nv, rbody, jnp.int32(0))
            return newcount, jnp.maximum(tau, _unkey(lo))

        def maybe_compact(count, tau):
            return lax.cond(count >= COMPACT_AT, compact,
                            lambda ct: ct, (count, tau))

        def append_vregs(xs, ibase, count, tau):
            for i, x in enumerate(xs):
                m = x > tau
                ones = jnp.where(m, 1, 0)
                pref = plsc.cumsum(ones)
                pos = count + pref - 1
                okm = m & (pos < CAP)
                pos = jnp.where(okm, pos, 0)
                plsc.store_scatter(candv, [pos], x, mask=okm)
                plsc.store_scatter(candi, [pos], ibase + 16 * i + lane,
                                   mask=okm)
                count = jnp.minimum(count + jnp.sum(ones), CAP)
            return maybe_compact(count, tau)

        def row_body(j, _):
            row = wid * RPW + j

            def dma_pair(c, dv, gv, sd, sg):
                return (pltpu.make_async_copy(
                            dist_hbm.at[row, pl.ds(c * CH, CH)], dv, sd),
                        pltpu.make_async_copy(
                            gum_hbm.at[row, pl.ds(c * CH, CH)], gv, sg))

            def start_chunk(c, dv, gv, sd, sg):
                a, b2 = dma_pair(c, dv, gv, sd, sg)
                a.start()
                b2.start()

            def wait_chunk(c, dv, gv, sd, sg):
                a, b2 = dma_pair(c, dv, gv, sd, sg)
                a.wait()
                b2.wait()

            def one_batch(dv, gv, gbase, base, count, tau, masked):
                xs = [dv[pl.ds(base + 16 * i, 16)]
                      + gv[pl.ds(base + 16 * i, 16)]
                      for i in range(BATCH)]
                if masked:
                    xs = [jnp.where(gbase + base + 16 * i + lane < V, x,
                                    jnp.float32(-jnp.inf))
                          for i, x in enumerate(xs)]
                acc = xs[0]
                for x in xs[1:]:
                    acc = jnp.maximum(acc, x)
                hit = jnp.max(acc) > tau
                return lax.cond(
                    hit,
                    lambda ct: append_vregs(xs, gbase + base, *ct),
                    lambda ct: ct, (count, tau))

            def proc(dv, gv, c, count, tau, nb=NBATCH):
                gbase = c * CH

                def batch_body(bb, ct):
                    return one_batch(dv, gv, gbase, bb * (16 * BATCH),
                                     *ct, masked=False)

                return lax.fori_loop(0, nb, batch_body, (count, tau))

            start_chunk(0, dist_v0, gum_v0, semd0, semg0)

            def pair_body(cc, ct):
                count, tau = ct
                c0 = 2 * cc
                wait_chunk(c0, dist_v0, gum_v0, semd0, semg0)
                start_chunk(c0 + 1, dist_v1, gum_v1, semd1, semg1)
                count, tau = proc(dist_v0, gum_v0, c0, count, tau)
                wait_chunk(c0 + 1, dist_v1, gum_v1, semd1, semg1)
                start_chunk(c0 + 2, dist_v0, gum_v0, semd0, semg0)
                return proc(dist_v1, gum_v1, c0 + 1, count, tau)

            count, tau = lax.fori_loop(
                0, NCH // 2, pair_body,
                (jnp.int32(0), jnp.float32(-jnp.inf)))

            cl = jnp.int32(NCH - 1)
            wait_chunk(cl, dist_v0, gum_v0, semd0, semg0)
            count, tau = proc(dist_v0, gum_v0, cl, count, tau, nb=NBATCH - 1)
            count, tau = one_batch(dist_v0, gum_v0, cl * CH,
                                   (NBATCH - 1) * 16 * BATCH, count, tau,
                                   masked=True)

            def bs2(_, lohi):
                lo, hi = lohi
                mid = lo + (hi - lo) // jnp.uint32(2)
                le = cnt_gt(mid, count) <= 63
                return (jnp.where(le, lo, mid + jnp.uint32(1)),
                        jnp.where(le, mid, hi))

            _, tstar = lax.fori_loop(
                0, 32, bs2, (jnp.uint32(0), jnp.uint32(0xFFFFFFFF)))
            cg_total = cnt_gt(tstar, count)
            obase = j * GROUP + 1
            nv = (count + 15) // 16

            def ebody(v, cnts):
                cg, ce = cnts
                val = candv[pl.ds(16 * v, 16)]
                ivv = candi[pl.ds(16 * v, 16)]
                key = _key16(val)
                valid = (lane + 16 * v) < count
                gtm = (key > tstar) & valid
                eqm = (key == tstar) & valid
                og = jnp.where(gtm, 1, 0)
                oe = jnp.where(eqm, 1, 0)
                pg = obase + cg + plsc.cumsum(og) - 1
                pe = obase + cg_total + ce + plsc.cumsum(oe) - 1
                pe_ok = eqm & (pe < obase + K)
                plsc.store_scatter(idxg, [jnp.where(gtm, pg, 0)], ivv,
                                   mask=gtm)
                plsc.store_scatter(idxg, [jnp.where(pe_ok, pe, 0)], ivv,
                                   mask=pe_ok)
                return cg + jnp.sum(og), ce + jnp.sum(oe)

            lax.fori_loop(0, nv, ebody, (jnp.int32(0), jnp.int32(0)))
            return 0

        lax.fori_loop(0, RPW, row_body, 0)
        pltpu.sync_copy(idxg, ids_hbm.at[pl.ds(wid * IPW_PAD, IPW_PAD)])

    return body(dist, gumbel, gt)


def _sc_gather(table, ids):
    mesh = plsc.VectorSubcoreMesh(core_axis_name="c", subcore_axis_name="s")

    @functools.partial(
        pl.kernel, mesh=mesh,
        out_type=jax.ShapeDtypeStruct((N_IDS, C), jnp.float32),
        compiler_params=pltpu.CompilerParams(use_tc_tiling_on_sc=False,
                                             needs_layout_passes=False),
        scratch_types=[
            pltpu.VMEM((IPW_PAD,), jnp.int32),
            pltpu.VMEM((IPW, C), jnp.float32),
            pltpu.SemaphoreType.DMA,
        ],
    )
    def gather_k(table_hbm, ids_hbm, out_hbm, idx_v, rows_v, sem):
        wid = lax.axis_index("s") * NC + lax.axis_index("c")
        pltpu.sync_copy(ids_hbm.at[pl.ds(wid * IPW_PAD, IPW_PAD)], idx_v)
        pltpu.async_copy(table_hbm.at[idx_v.at[pl.ds(0, IPW)]], rows_v,
                         sem).wait()
        pltpu.sync_copy(rows_v, out_hbm.at[pl.ds(wid * IPW, IPW)])

    return gather_k(table, ids)


def _mlp_body(emb_ref, rows_ref, w1e_ref, w1c_ref, b1_ref, w2_ref, b2_ref,
              out_ref):
    emb = emb_ref[...]
    w1e = w1e_ref[...]
    w1c = w1c_ref[...]
    b1 = b1_ref[...]
    w2 = w2_ref[...]
    b2 = b2_ref[0, 0]

    a = jnp.dot(emb, w1e, preferred_element_type=jnp.float32)
    ctx = rows_ref[...]
    cc = jnp.dot(ctx, w1c, preferred_element_type=jnp.float32)
    a_rep = jnp.broadcast_to(a[:, None, :], (B, GROUP, H)).reshape(N_IDS, H)
    h = jnp.maximum(a_rep + cc + b1, 0.0)
    s = jnp.dot(h, w2, preferred_element_type=jnp.float32) + b2

    ridx = lax.broadcasted_iota(jnp.int32, (N_IDS, 1), 0)
    y = jnp.where(ridx % GROUP == 0, 1.0, 0.0)
    t = jnp.maximum(s, 0.0) - s * y + jnp.log(1.0 + jnp.exp(-jnp.abs(s)))
    out_ref[...] = jnp.reshape(jnp.sum(t) / jnp.float32(N_IDS), (1, 1))


def _mlp_loss(embedding, rows, W1, b1, W2, b2):
    out = pl.pallas_call(
        _mlp_body,
        out_shape=jax.ShapeDtypeStruct((1, 1), jnp.float32),
    )(embedding, rows, W1[:E], W1[E:], b1.reshape(1, H), W2, b2.reshape(1, 1))
    return out.reshape(())


def kernel(embedding, gt_token, next_token_dist, emb_table, W1, b1, W2, b2):
    ids = _sc_topk(next_token_dist, _GUMBEL, gt_token.astype(jnp.int32))
    ctx = _sc_gather(emb_table, ids)
    return _mlp_loss(embedding, ctx, W1, b1, W2, b2)
